# Initial kernel scaffold; baseline (speedup 1.0000x reference)
import jax
import jax.numpy as jnp
from jax.experimental import pallas as pl


def kernel(x, edge_index, W1l, b1l, W1r, W2l, b2l, W2r):
    raise NotImplementedError("write your pallas kernel here")


# speedup vs baseline: 20.8689x; 20.8689x over previous
import functools

import jax
import jax.numpy as jnp
from jax import lax
from jax.experimental import pallas as pl
from jax.experimental.pallas import tpu as pltpu
from jax.experimental.pallas import tpu_sc as plsc

N_NODES_P = 10240
_NC, _NS = 2, 16
_NW = _NC * _NS

_mesh = plsc.VectorSubcoreMesh(core_axis_name="c", subcore_axis_name="s")


def _mm_body(x_ref, wl_ref, wr_ref, y_ref, r_ref):
    xv = x_ref[...]
    dims = (((1,), (1,)), ((), ()))
    y_ref[...] = lax.dot_general(xv, wl_ref[...], dims,
                                 preferred_element_type=jnp.float32)
    r_ref[...] = lax.dot_general(xv, wr_ref[...], dims,
                                 preferred_element_type=jnp.float32)


def _mid_body(a0_ref, a1_ref, c0_ref, c1_ref, r1_ref, b1_ref, w2l_ref,
              w2r_ref, z_ref, r2_ref):
    cnt = jnp.clip(c0_ref[...] + c1_ref[...], 1.0, None)
    agg = a0_ref[...] + a1_ref[...]
    h = jnp.maximum(agg / cnt + b1_ref[...] + r1_ref[...], 0.0)
    z_ref[...] = jnp.sum(h * w2l_ref[...], axis=1, keepdims=True)
    r2_ref[...] = jnp.sum(h * w2r_ref[...], axis=1, keepdims=True)


def _out_body(a0_ref, a1_ref, c0_ref, c1_ref, r2_ref, b2_ref, o_ref):
    cnt = jnp.clip(c0_ref[...] + c1_ref[...], 1.0, None)
    t = (a0_ref[...] + a1_ref[...]) / cnt + b2_ref[...] + r2_ref[...]
    o_ref[...] = jax.nn.sigmoid(t)


_CHUNK = 2000


def _edge1_body(y1, src, dst, zag, zcnt, agg_p, cnt_p,
                agg_sh, cnt_sh, src_v, dst_v, rows_v, ones_v, sem):
    c = lax.axis_index("c")
    s = lax.axis_index("s")
    rows_per_s = N_NODES_P // _NS

    def _fill(g, _):
        ones_v[pl.ds(g * 16, 16)] = jnp.ones((16,), jnp.float32)
        return 0
    lax.fori_loop(0, _CHUNK // 16, _fill, 0)

    pltpu.sync_copy(zag.at[pl.ds(s * rows_per_s, rows_per_s)],
                    agg_sh.at[pl.ds(s * rows_per_s, rows_per_s)])
    pltpu.sync_copy(zcnt.at[pl.ds(s * rows_per_s, rows_per_s)],
                    cnt_sh.at[pl.ds(s * rows_per_s, rows_per_s)])
    plsc.subcore_barrier()

    w = s * _NC + c
    e_per_w = src.shape[0] // _NW
    base_w = w * e_per_w

    def _step(i, _):
        base = pl.multiple_of(base_w + i * _CHUNK, 8)
        pltpu.sync_copy(src.at[pl.ds(base, _CHUNK)], src_v)
        pltpu.async_copy(y1.at[src_v], rows_v, sem).wait()
        pltpu.sync_copy(dst.at[pl.ds(base, _CHUNK)], dst_v)
        pltpu.sync_copy(rows_v, agg_sh.at[dst_v], add=True)
        pltpu.sync_copy(ones_v, cnt_sh.at[dst_v], add=True)
        return 0
    lax.fori_loop(0, e_per_w // _CHUNK, _step, 0)

    plsc.subcore_barrier()
    pltpu.sync_copy(agg_sh.at[pl.ds(s * rows_per_s, rows_per_s)],
                    agg_p.at[c, pl.ds(s * rows_per_s, rows_per_s)])
    pltpu.sync_copy(cnt_sh.at[pl.ds(s * rows_per_s, rows_per_s)],
                    cnt_p.at[c, pl.ds(s * rows_per_s, rows_per_s)])


def _edge2_body(z, src, dst, zcnt, agg_p,
                agg_sh, z_v, src_v, dst_v, gath_v, sem):
    c = lax.axis_index("c")
    s = lax.axis_index("s")
    rows_per_s = N_NODES_P // _NS

    pltpu.sync_copy(zcnt.at[pl.ds(s * rows_per_s, rows_per_s)],
                    agg_sh.at[pl.ds(s * rows_per_s, rows_per_s)])
    pltpu.sync_copy(z, z_v)
    plsc.subcore_barrier()

    w = s * _NC + c
    e_per_w = src.shape[0] // _NW
    base_w = w * e_per_w

    def _step(i, _):
        base = pl.multiple_of(base_w + i * _CHUNK, 8)
        pltpu.sync_copy(src.at[pl.ds(base, _CHUNK)], src_v)

        def _g(g, _):
            idx = src_v[pl.ds(g * 16, 16)]
            gath_v[pl.ds(g * 16, 16)] = plsc.load_gather(z_v, [idx])
            return 0
        lax.fori_loop(0, _CHUNK // 16, _g, 0)

        pltpu.sync_copy(dst.at[pl.ds(base, _CHUNK)], dst_v)
        pltpu.sync_copy(gath_v, agg_sh.at[dst_v], add=True)
        return 0
    lax.fori_loop(0, e_per_w // _CHUNK, _step, 0)

    plsc.subcore_barrier()
    pltpu.sync_copy(agg_sh.at[pl.ds(s * rows_per_s, rows_per_s)],
                    agg_p.at[c, pl.ds(s * rows_per_s, rows_per_s)])


def kernel(x, edge_index, W1l, b1l, W1r, W2l, b2l, W2r):
    n, d_in = x.shape
    d_hid = W1l.shape[0]
    e = edge_index.shape[1]
    ei = edge_index.astype(jnp.int32)
    src, dst = ei[0], ei[1]

    y1, r1 = pl.pallas_call(
        _mm_body,
        out_shape=(jax.ShapeDtypeStruct((n, d_hid), jnp.float32),
                   jax.ShapeDtypeStruct((n, d_hid), jnp.float32)),
    )(x, W1l, W1r)

    zag = jnp.zeros((N_NODES_P, d_hid), jnp.float32)
    zcnt = jnp.zeros((N_NODES_P,), jnp.float32)

    edge1 = pl.kernel(
        _edge1_body,
        out_type=(jax.ShapeDtypeStruct((_NC, N_NODES_P, d_hid), jnp.float32),
                  jax.ShapeDtypeStruct((_NC, N_NODES_P), jnp.float32)),
        mesh=_mesh,
        compiler_params=pltpu.CompilerParams(use_tc_tiling_on_sc=False),
        scratch_types=[
            pltpu.VMEM_SHARED((N_NODES_P, d_hid), jnp.float32),
            pltpu.VMEM_SHARED((N_NODES_P,), jnp.float32),
            pltpu.VMEM((_CHUNK,), jnp.int32),
            pltpu.VMEM((_CHUNK,), jnp.int32),
            pltpu.VMEM((_CHUNK, d_hid), jnp.float32),
            pltpu.VMEM((_CHUNK,), jnp.float32),
            pltpu.SemaphoreType.DMA,
        ],
    )
    agg_p, cnt_p = edge1(y1, src, dst, zag, zcnt)

    a0, a1 = agg_p[0, :n], agg_p[1, :n]
    c0, c1 = cnt_p[0, :n, None], cnt_p[1, :n, None]

    z, r2 = pl.pallas_call(
        _mid_body,
        out_shape=(jax.ShapeDtypeStruct((n, 1), jnp.float32),
                   jax.ShapeDtypeStruct((n, 1), jnp.float32)),
    )(a0, a1, c0, c1, r1, b1l[None, :], W2l, W2r)

    edge2 = pl.kernel(
        _edge2_body,
        out_type=jax.ShapeDtypeStruct((_NC, N_NODES_P), jnp.float32),
        mesh=_mesh,
        compiler_params=pltpu.CompilerParams(use_tc_tiling_on_sc=False,
                                             needs_layout_passes=False),
        scratch_types=[
            pltpu.VMEM_SHARED((N_NODES_P,), jnp.float32),
            pltpu.VMEM((n,), jnp.float32),
            pltpu.VMEM((_CHUNK,), jnp.int32),
            pltpu.VMEM((_CHUNK,), jnp.int32),
            pltpu.VMEM((_CHUNK,), jnp.float32),
            pltpu.SemaphoreType.DMA,
        ],
    )
    agg2_p = edge2(z[:, 0], src, dst, zcnt)

    g0, g1 = agg2_p[0, :n, None], agg2_p[1, :n, None]

    out = pl.pallas_call(
        _out_body,
        out_shape=jax.ShapeDtypeStruct((n, 1), jnp.float32),
    )(g0, g1, c0, c1, r2, b2l[None, :])

    return out[:, 0]


# speedup vs baseline: 25.0937x; 1.2024x over previous
import jax
import jax.numpy as jnp
from jax import lax
from jax.experimental import pallas as pl
from jax.experimental.pallas import tpu as pltpu
from jax.experimental.pallas import tpu_sc as plsc

N = 10000
NP = 10240
_NC, _NS = 2, 16
_NW = _NC * _NS

_mesh = plsc.VectorSubcoreMesh(core_axis_name="c", subcore_axis_name="s")
_sc_params = pltpu.CompilerParams(use_tc_tiling_on_sc=False,
                                  needs_layout_passes=False)

_CHUNK = 2000


def _mm_body(x_ref, wl_ref, wr_ref, y_ref, r_ref):
    xv = x_ref[...]
    dims = (((1,), (1,)), ((), ()))
    y_ref[0:N, :] = lax.dot_general(xv, wl_ref[...], dims,
                                    preferred_element_type=jnp.float32)
    r_ref[0:N, :] = lax.dot_general(xv, wr_ref[...], dims,
                                    preferred_element_type=jnp.float32)
    pad = jnp.zeros((NP - N, wl_ref.shape[0]), jnp.float32)
    y_ref[N:NP, :] = pad
    r_ref[N:NP, :] = pad


def _fill(ref, n16, value):
    def body(g, _):
        ref[pl.ds(g * 16, 16)] = jnp.full((16,), value, jnp.float32)
        return 0
    lax.fori_loop(0, n16, body, 0)


def _fill2(ref, nrows, ncols, value):
    def body(r, _):
        for j in range(ncols // 16):
            ref[r, pl.ds(j * 16, 16)] = jnp.full((16,), value, jnp.float32)
        return 0
    lax.fori_loop(0, nrows, body, 0)


def _edge1_body(y1, ei, agg_p, cnt_p,
                agg_sh, cnt_sh, src_v, dst_v, rows_v, ones_v,
                zero_v, zero1_v, sem):
    c = lax.axis_index("c")
    s = lax.axis_index("s")
    rps = NP // _NS

    _fill(ones_v, _CHUNK // 16, 1.0)
    _fill2(zero_v, rps, 32, 0.0)
    _fill(zero1_v, rps // 16, 0.0)
    pltpu.sync_copy(zero_v, agg_sh.at[pl.ds(s * rps, rps)])
    pltpu.sync_copy(zero1_v, cnt_sh.at[pl.ds(s * rps, rps)])
    plsc.subcore_barrier()

    w = s * _NC + c
    e_per_w = ei.shape[1] // _NW
    base_w = w * e_per_w

    def _step(i, _):
        base = pl.multiple_of(base_w + i * _CHUNK, 8)
        pltpu.sync_copy(ei.at[0, pl.ds(base, _CHUNK)], src_v)
        pltpu.async_copy(y1.at[src_v], rows_v, sem).wait()
        pltpu.sync_copy(ei.at[1, pl.ds(base, _CHUNK)], dst_v)
        pltpu.sync_copy(rows_v, agg_sh.at[dst_v], add=True)
        pltpu.sync_copy(ones_v, cnt_sh.at[dst_v], add=True)
        return 0
    lax.fori_loop(0, e_per_w // _CHUNK, _step, 0)

    plsc.subcore_barrier()
    pltpu.sync_copy(agg_sh.at[pl.ds(s * rps, rps)],
                    agg_p.at[c, pl.ds(s * rps, rps)])
    pltpu.sync_copy(cnt_sh.at[pl.ds(s * rps, rps)],
                    cnt_p.at[c, pl.ds(s * rps, rps)])


def _mid_edge2_body(agg_p, cnt_p, r1, b1, w2l, w2r, ei,
                    agg2_p, r2_out,
                    z_sh, agg2_sh,
                    a0_v, a1_v, c0_v, c1_v, r1_v, b1_v, w2l_v, w2r_v,
                    zloc_v, r2loc_v, z_v, src_v, dst_v, gath_v, zero1_v,
                    sem):
    c = lax.axis_index("c")
    s = lax.axis_index("s")
    rps = NP // _NS
    r0 = s * rps

    pltpu.sync_copy(agg_p.at[0, pl.ds(r0, rps)], a0_v)
    pltpu.sync_copy(agg_p.at[1, pl.ds(r0, rps)], a1_v)
    pltpu.sync_copy(cnt_p.at[0, pl.ds(r0, rps)], c0_v)
    pltpu.sync_copy(cnt_p.at[1, pl.ds(r0, rps)], c1_v)
    pltpu.sync_copy(r1.at[pl.ds(r0, rps)], r1_v)
    pltpu.sync_copy(b1, b1_v)
    pltpu.sync_copy(w2l, w2l_v)
    pltpu.sync_copy(w2r, w2r_v)
    _fill(zero1_v, rps // 16, 0.0)

    lanes = lax.broadcasted_iota(jnp.int32, (16,), 0)

    def _grp(g, _):
        row0 = g * 16
        rows = row0 + lanes
        cnt = jnp.maximum(c0_v[pl.ds(row0, 16)] + c1_v[pl.ds(row0, 16)], 1.0)
        z16 = jnp.zeros((16,), jnp.float32)
        r216 = jnp.zeros((16,), jnp.float32)
        for k in range(32):
            cols = jnp.full((16,), k, jnp.int32)
            acol = (plsc.load_gather(a0_v, [rows, cols]) +
                    plsc.load_gather(a1_v, [rows, cols]))
            rcol = plsc.load_gather(r1_v, [rows, cols])
            h = jnp.maximum(acol / cnt + b1_v[k, pl.ds(0, 16)] + rcol, 0.0)
            z16 = z16 + h * w2l_v[k, pl.ds(0, 16)]
            r216 = r216 + h * w2r_v[k, pl.ds(0, 16)]
        zloc_v[pl.ds(row0, 16)] = z16
        r2loc_v[pl.ds(row0, 16)] = r216
        return 0
    lax.fori_loop(0, rps // 16, _grp, 0)

    pltpu.sync_copy(zloc_v, z_sh.at[pl.ds(r0, rps)])
    pltpu.sync_copy(zero1_v, agg2_sh.at[pl.ds(r0, rps)])

    @pl.when(c == 0)
    def _():
        pltpu.sync_copy(r2loc_v, r2_out.at[pl.ds(r0, rps)])

    plsc.subcore_barrier()

    pltpu.sync_copy(z_sh, z_v)

    w = s * _NC + c
    e_per_w = ei.shape[1] // _NW
    base_w = w * e_per_w

    def _step(i, _):
        base = pl.multiple_of(base_w + i * _CHUNK, 8)
        pltpu.sync_copy(ei.at[0, pl.ds(base, _CHUNK)], src_v)

        def _g(g, _):
            idx = src_v[pl.ds(g * 16, 16)]
            gath_v[pl.ds(g * 16, 16)] = plsc.load_gather(z_v, [idx])
            return 0
        lax.fori_loop(0, _CHUNK // 16, _g, 0)

        pltpu.sync_copy(ei.at[1, pl.ds(base, _CHUNK)], dst_v)
        pltpu.sync_copy(gath_v, agg2_sh.at[dst_v], add=True)
        return 0
    lax.fori_loop(0, e_per_w // _CHUNK, _step, 0)

    plsc.subcore_barrier()
    pltpu.sync_copy(agg2_sh.at[pl.ds(r0, rps)],
                    agg2_p.at[c, pl.ds(r0, rps)])


def _out_body(agg2_p, cnt_p, r2, b2,
              o_hbm,
              g0_v, g1_v, c0_v, c1_v, r2_v, b2_v, o_v, sem):
    c = lax.axis_index("c")
    s = lax.axis_index("s")
    rps = NP // _NW
    t = s * _NC + c
    r0 = t * rps

    pltpu.sync_copy(agg2_p.at[0, pl.ds(r0, rps)], g0_v)
    pltpu.sync_copy(agg2_p.at[1, pl.ds(r0, rps)], g1_v)
    pltpu.sync_copy(cnt_p.at[0, pl.ds(r0, rps)], c0_v)
    pltpu.sync_copy(cnt_p.at[1, pl.ds(r0, rps)], c1_v)
    pltpu.sync_copy(r2.at[pl.ds(r0, rps)], r2_v)
    pltpu.sync_copy(b2, b2_v)

    b2 = b2_v[...]

    def _grp(g, _):
        d = pl.ds(g * 16, 16)
        cnt = jnp.maximum(c0_v[d] + c1_v[d], 1.0)
        t16 = (g0_v[d] + g1_v[d]) / cnt + b2 + r2_v[d]
        o_v[d] = 1.0 / (1.0 + jnp.exp(-t16))
        return 0
    lax.fori_loop(0, rps // 16, _grp, 0)

    pltpu.sync_copy(o_v, o_hbm.at[pl.ds(r0, rps)])


def kernel(x, edge_index, W1l, b1l, W1r, W2l, b2l, W2r):
    d_hid = W1l.shape[0]
    ei = edge_index.astype(jnp.int32)

    y1, r1 = pl.pallas_call(
        _mm_body,
        out_shape=(jax.ShapeDtypeStruct((NP, d_hid), jnp.float32),
                   jax.ShapeDtypeStruct((NP, d_hid), jnp.float32)),
    )(x, W1l, W1r)

    agg_p, cnt_p = pl.kernel(
        _edge1_body,
        out_type=(jax.ShapeDtypeStruct((_NC, NP, d_hid), jnp.float32),
                  jax.ShapeDtypeStruct((_NC, NP), jnp.float32)),
        mesh=_mesh,
        compiler_params=_sc_params,
        scratch_types=[
            pltpu.VMEM_SHARED((NP, d_hid), jnp.float32),
            pltpu.VMEM_SHARED((NP,), jnp.float32),
            pltpu.VMEM((_CHUNK,), jnp.int32),
            pltpu.VMEM((_CHUNK,), jnp.int32),
            pltpu.VMEM((_CHUNK, d_hid), jnp.float32),
            pltpu.VMEM((_CHUNK,), jnp.float32),
            pltpu.VMEM((NP // _NS, d_hid), jnp.float32),
            pltpu.VMEM((NP // _NS,), jnp.float32),
            pltpu.SemaphoreType.DMA,
        ],
    )(y1, ei)

    agg2_p, r2 = pl.kernel(
        _mid_edge2_body,
        out_type=(jax.ShapeDtypeStruct((_NC, NP), jnp.float32),
                  jax.ShapeDtypeStruct((NP,), jnp.float32)),
        mesh=_mesh,
        compiler_params=_sc_params,
        scratch_types=[
            pltpu.VMEM_SHARED((NP,), jnp.float32),
            pltpu.VMEM_SHARED((NP,), jnp.float32),
            pltpu.VMEM((NP // _NS, d_hid), jnp.float32),
            pltpu.VMEM((NP // _NS, d_hid), jnp.float32),
            pltpu.VMEM((NP // _NS,), jnp.float32),
            pltpu.VMEM((NP // _NS,), jnp.float32),
            pltpu.VMEM((NP // _NS, d_hid), jnp.float32),
            pltpu.VMEM((d_hid, 16), jnp.float32),
            pltpu.VMEM((d_hid, 16), jnp.float32),
            pltpu.VMEM((d_hid, 16), jnp.float32),
            pltpu.VMEM((NP // _NS,), jnp.float32),
            pltpu.VMEM((NP // _NS,), jnp.float32),
            pltpu.VMEM((NP,), jnp.float32),
            pltpu.VMEM((_CHUNK,), jnp.int32),
            pltpu.VMEM((_CHUNK,), jnp.int32),
            pltpu.VMEM((_CHUNK,), jnp.float32),
            pltpu.VMEM((NP // _NS,), jnp.float32),
            pltpu.SemaphoreType.DMA,
        ],
    )(agg_p, cnt_p, r1,
      jnp.broadcast_to(b1l[:, None], (d_hid, 16)),
      jnp.broadcast_to(W2l[0][:, None], (d_hid, 16)),
      jnp.broadcast_to(W2r[0][:, None], (d_hid, 16)), ei)

    o = pl.kernel(
        _out_body,
        out_type=jax.ShapeDtypeStruct((NP,), jnp.float32),
        mesh=_mesh,
        compiler_params=_sc_params,
        scratch_types=[
            pltpu.VMEM((NP // _NW,), jnp.float32),
            pltpu.VMEM((NP // _NW,), jnp.float32),
            pltpu.VMEM((NP // _NW,), jnp.float32),
            pltpu.VMEM((NP // _NW,), jnp.float32),
            pltpu.VMEM((NP // _NW,), jnp.float32),
            pltpu.VMEM((16,), jnp.float32),
            pltpu.VMEM((NP // _NW,), jnp.float32),
            pltpu.SemaphoreType.DMA,
        ],
    )(agg2_p, cnt_p, r2, jnp.broadcast_to(b2l, (16,)))

    return o[:N]
